# baseline (device time: 27463 ns/iter reference)
import jax
import jax.numpy as jnp
from jax import lax
from jax.experimental import pallas as pl
from jax.experimental.pallas import tpu as pltpu

N_DEV = 4
N_LAYERS = 3
NS = 4


def kernel(x, Win0, Wout0, Win1, Wout1, Win2, Wout2):
    b, d_in = x.shape
    h_dim = Win0.shape[1]
    out_cols = Wout0.shape[1]
    bh = b // NS

    def body(x_hbm, win0, wout0, win1, wout1, win2, wout2, out_hbm,
             comm_ref, send_sems, recv_sems,
             x_vmem, win_vmem, wout_vmem, out_stage, local_sems):
        my = lax.axis_index("i")

        cx = pltpu.make_async_copy(x_hbm, x_vmem, local_sems.at[0])
        cx.start()
        wcopies = []
        for l, (wi, wo) in enumerate(
            ((win0, wout0), (win1, wout1), (win2, wout2))
        ):
            ci = pltpu.make_async_copy(
                wi, win_vmem.at[l], local_sems.at[1 + 2 * l])
            ci.start()
            co = pltpu.make_async_copy(
                wo, wout_vmem.at[l], local_sems.at[2 + 2 * l])
            co.start()
            wcopies.append((ci, co))

        barrier_sem = pltpu.get_barrier_semaphore()
        for d in range(1, N_DEV):
            pl.semaphore_signal(
                barrier_sem, inc=1,
                device_id=((my + d) % N_DEV,),
                device_id_type=pl.DeviceIdType.MESH,
            )
        pl.semaphore_wait(barrier_sem, N_DEV - 1)

        rdmas = {}

        def compute_send(l, h, x_slice):
            p = jnp.dot(
                x_slice, win_vmem[l, :, :],
                preferred_element_type=jnp.float32,
            )
            comm_ref[l, h, 0, :, :] = p.astype(jnp.bfloat16)
            for d in (2, 1, 3):
                r = pltpu.make_async_remote_copy(
                    src_ref=comm_ref.at[l, h, 0],
                    dst_ref=comm_ref.at[l, h, d],
                    send_sem=send_sems.at[l, h, d - 1],
                    recv_sem=recv_sems.at[l, h, d - 1],
                    device_id=((my + d) % N_DEV,),
                    device_id_type=pl.DeviceIdType.MESH,
                )
                r.start()
                rdmas[(l, h, d)] = r

        def recv_mm2(l, h):
            acc = comm_ref[l, h, 0, :, :].astype(jnp.float32)
            for d in (1, 3, 2):
                rdmas[(l, h, d)].wait_recv()
                acc = acc + comm_ref[l, h, d, :, :].astype(jnp.float32)
            hidden = jnp.maximum(acc, 0.0)
            return jnp.dot(
                hidden, wout_vmem[l, :, :],
                preferred_element_type=jnp.float32,
            )

        cx.wait()
        wcopies[0][0].wait()
        for h in range(NS):
            compute_send(0, h, x_vmem[h * bh:(h + 1) * bh, :])
        for l in range(N_LAYERS - 1):
            wcopies[l][1].wait()
            wcopies[l + 1][0].wait()
            for h in range(NS):
                compute_send(l + 1, h, recv_mm2(l, h))
        wcopies[N_LAYERS - 1][1].wait()
        for h in range(NS):
            out_stage[h * bh:(h + 1) * bh, :] = recv_mm2(N_LAYERS - 1, h)

        cout = pltpu.make_async_copy(out_stage, out_hbm, local_sems.at[7])
        cout.start()
        cout.wait()

        for r in rdmas.values():
            r.wait_send()

    return pl.pallas_call(
        body,
        out_shape=jax.ShapeDtypeStruct((b, out_cols), jnp.float32),
        in_specs=[pl.BlockSpec(memory_space=pl.ANY)] * 7,
        out_specs=pl.BlockSpec(memory_space=pl.ANY),
        scratch_shapes=[
            pltpu.VMEM((N_LAYERS, NS, N_DEV, bh, h_dim), jnp.bfloat16),
            pltpu.SemaphoreType.DMA((N_LAYERS, NS, N_DEV - 1)),
            pltpu.SemaphoreType.DMA((N_LAYERS, NS, N_DEV - 1)),
            pltpu.VMEM((b, d_in), jnp.float32),
            pltpu.VMEM((N_LAYERS, d_in, h_dim), jnp.float32),
            pltpu.VMEM((N_LAYERS, h_dim, out_cols), jnp.float32),
            pltpu.VMEM((b, out_cols), jnp.float32),
            pltpu.SemaphoreType.DMA((8,)),
        ],
        compiler_params=pltpu.CompilerParams(collective_id=0),
    )(x, Win0, Wout0, Win1, Wout1, Win2, Wout2)


# device time: 26339 ns/iter; 1.0427x vs baseline; 1.0427x over previous
import jax
import jax.numpy as jnp
from jax import lax
from jax.experimental import pallas as pl
from jax.experimental.pallas import tpu as pltpu

N_DEV = 4
N_LAYERS = 3
N_HALF = 4


def kernel(x, Win0, Wout0, Win1, Wout1, Win2, Wout2):
    b, _ = x.shape
    h_dim = Win0.shape[1]
    out_cols = Wout0.shape[1]
    bh = b // N_HALF

    def body(x_ref, win0, wout0, win1, wout1, win2, wout2,
             out_ref, comm_ref, send_sems, recv_sems):
        my = lax.axis_index("i")

        barrier_sem = pltpu.get_barrier_semaphore()
        for d in range(1, N_DEV):
            pl.semaphore_signal(
                barrier_sem, inc=1,
                device_id=((my + d) % N_DEV,),
                device_id_type=pl.DeviceIdType.MESH,
            )
        pl.semaphore_wait(barrier_sem, N_DEV - 1)

        wins = (win0, win1, win2)
        wouts = (wout0, wout1, wout2)
        rdmas = {}

        def compute_send(l, h, x_half):
            p = jnp.dot(
                x_half, wins[l][:, :],
                preferred_element_type=jnp.float32,
            )
            comm_ref[l, h, 0, :, :] = p.astype(jnp.bfloat16)
            for d in (2, 1, 3):
                r = pltpu.make_async_remote_copy(
                    src_ref=comm_ref.at[l, h, 0],
                    dst_ref=comm_ref.at[l, h, d],
                    send_sem=send_sems.at[l, h, d - 1],
                    recv_sem=recv_sems.at[l, h, d - 1],
                    device_id=((my + d) % N_DEV,),
                    device_id_type=pl.DeviceIdType.MESH,
                )
                r.start()
                rdmas[(l, h, d)] = r

        def recv_mm2(l, h):
            acc = comm_ref[l, h, 0, :, :].astype(jnp.float32)
            for d in (1, 3, 2):
                rdmas[(l, h, d)].wait_recv()
                acc = acc + comm_ref[l, h, d, :, :].astype(jnp.float32)
            hidden = jnp.maximum(acc, 0.0)
            return jnp.dot(
                hidden, wouts[l][:, :],
                preferred_element_type=jnp.float32,
            )

        for h in range(N_HALF):
            compute_send(0, h, x_ref[h * bh:(h + 1) * bh, :])
        for l in range(N_LAYERS - 1):
            for h in range(N_HALF):
                compute_send(l + 1, h, recv_mm2(l, h))
        for h in range(N_HALF):
            out_ref[h * bh:(h + 1) * bh, :] = recv_mm2(N_LAYERS - 1, h)

        for r in rdmas.values():
            r.wait_send()

    return pl.pallas_call(
        body,
        out_shape=jax.ShapeDtypeStruct((b, out_cols), jnp.float32),
        in_specs=[pl.BlockSpec(memory_space=pltpu.VMEM)] * 7,
        out_specs=pl.BlockSpec(memory_space=pltpu.VMEM),
        scratch_shapes=[
            pltpu.VMEM((N_LAYERS, N_HALF, N_DEV, bh, h_dim), jnp.bfloat16),
            pltpu.SemaphoreType.DMA((N_LAYERS, N_HALF, N_DEV - 1)),
            pltpu.SemaphoreType.DMA((N_LAYERS, N_HALF, N_DEV - 1)),
        ],
        compiler_params=pltpu.CompilerParams(collective_id=0),
    )(x, Win0, Wout0, Win1, Wout1, Win2, Wout2)
